# baseline (speedup 1.0000x reference)
import functools
import jax
import jax.numpy as jnp
from jax import lax
from jax.experimental import pallas as pl
from jax.experimental.pallas import tpu as pltpu
from jax.experimental.pallas import tpu_sc as plsc

_RATIO = 0.25
_R2 = 0.2 * 0.2
_K = 64


def _b16(v):
    return v.astype(jnp.bfloat16).astype(jnp.float32)


def _fps_body(px_ref, py_ref, pz_ref, idx_ref, dists_ref, rm_ref, *, m):
    px = px_ref[:]
    py = py_ref[:]
    pz = pz_ref[:]
    rows, cols = px.shape
    col1 = lax.broadcasted_iota(jnp.int32, (1, cols), 1)
    row1 = lax.broadcasted_iota(jnp.int32, (rows, 1), 0)

    px0 = px_ref[0, 0]
    py0 = py_ref[0, 0]
    pz0 = pz_ref[0, 0]
    d0 = (px - px0) ** 2 + (py - py0) ** 2 + (pz - pz0) ** 2
    dists_ref[:] = d0
    rm_ref[:] = jnp.max(d0, axis=1, keepdims=True)
    idx_ref[0:1, 0:1] = jnp.zeros((1, 1), jnp.int32)
    bigi = jnp.int32(2**30)

    def body(i, _):
        rm = rm_ref[:]
        mx = jnp.max(rm)
        r = jnp.min(jnp.where(rm == mx, row1, bigi))
        drow = dists_ref[pl.ds(r, 1), :]
        c = jnp.min(jnp.where(drow == mx, col1, bigi))
        nxt = r * cols + c
        cmask = col1 == c
        pxv = jnp.sum(jnp.where(cmask, px_ref[pl.ds(r, 1), :], 0.0))
        pyv = jnp.sum(jnp.where(cmask, py_ref[pl.ds(r, 1), :], 0.0))
        pzv = jnp.sum(jnp.where(cmask, pz_ref[pl.ds(r, 1), :], 0.0))
        dn = (px - pxv) ** 2 + (py - pyv) ** 2 + (pz - pzv) ** 2
        dnew = jnp.minimum(dists_ref[:], dn)
        dists_ref[:] = dnew
        rm_ref[:] = jnp.max(dnew, axis=1, keepdims=True)
        idx_ref[pl.ds(i, 1), :] = nxt.reshape(1, 1)
        return 0

    lax.fori_loop(1, m, body, 0)


def _fps(pos, m, interpret=False):
    n = pos.shape[0]
    rows = n // 128
    px = pos[:, 0].reshape(rows, 128)
    py = pos[:, 1].reshape(rows, 128)
    pz = pos[:, 2].reshape(rows, 128)
    idx = pl.pallas_call(
        functools.partial(_fps_body, m=m),
        out_shape=jax.ShapeDtypeStruct((m, 1), jnp.int32),
        scratch_shapes=[pltpu.VMEM((rows, 128), jnp.float32),
                        pltpu.VMEM((rows, 1), jnp.float32)],
        interpret=interpret,
    )(px, py, pz)
    return idx.reshape(m)


def _feat_body(p_ref, q_ref, w_ref, b_ref, a_ref, c_ref):
    w = _b16(w_ref[:])
    a_ref[:] = jnp.dot(_b16(p_ref[:]), w,
                       preferred_element_type=jnp.float32) + b_ref[:]
    c_ref[:] = jnp.dot(_b16(q_ref[:]), w, preferred_element_type=jnp.float32)


def _features(x, pos, pos_c, W1, b1, interpret=False):
    n = x.shape[0]
    m = pos_c.shape[0]
    dh = W1.shape[1]
    p = jnp.concatenate([x, pos, jnp.zeros((n, 128 - 9), jnp.float32)], axis=1)
    q = jnp.concatenate(
        [jnp.zeros((m, 6), jnp.float32), pos_c, jnp.zeros((m, 128 - 9), jnp.float32)],
        axis=1)
    w = jnp.concatenate([W1, jnp.zeros((128 - 9, dh), jnp.float32)], axis=0)
    bn = 1024
    bm = bn * m // n
    grid = n // bn
    return pl.pallas_call(
        _feat_body,
        grid=(grid,),
        in_specs=[
            pl.BlockSpec((bn, 128), lambda g: (g, 0)),
            pl.BlockSpec((bm, 128), lambda g: (g, 0)),
            pl.BlockSpec((128, dh), lambda g: (0, 0)),
            pl.BlockSpec((1, dh), lambda g: (0, 0)),
        ],
        out_specs=(pl.BlockSpec((bn, dh), lambda g: (g, 0)),
                   pl.BlockSpec((bm, dh), lambda g: (g, 0))),
        out_shape=(jax.ShapeDtypeStruct((n, dh), jnp.float32),
                   jax.ShapeDtypeStruct((m, dh), jnp.float32)),
        interpret=interpret,
    )(p, q, w, b1.reshape(1, dh))


def _select_body(cx_ref, cy_ref, cz_ref, px_ref, py_ref, pz_ref,
                 nbrt_ref, cnt_ref, d2_ref, *, n):
    bm = cx_ref.shape[0]
    cx = cx_ref[:]
    cy = cy_ref[:]
    cz = cz_ref[:]
    px = px_ref[:]
    py = py_ref[:]
    pz = pz_ref[:]
    sc = (cx * cx + cy * cy) + cz * cz
    sp = (px * px + py * py) + pz * pz
    dot = (_b16(cx) * _b16(px) + _b16(cy) * _b16(py)) + _b16(cz) * _b16(pz)
    d2_ref[:] = jnp.maximum((sc + sp) - 2.0 * dot, 0.0)
    colv = lax.broadcasted_iota(jnp.int32, (bm, n), 1)
    big = jnp.float32(3e38)

    def body(k, cnt):
        d2 = d2_ref[:]
        v = jnp.min(d2, axis=1, keepdims=True)
        j = jnp.min(jnp.where(d2 == v, colv, jnp.int32(2**30)), axis=1,
                    keepdims=True)
        nbrt_ref[0:1, pl.ds(k, 1), :] = j.reshape(1, 1, bm)
        d2_ref[:] = jnp.where(colv == j, big, d2)
        return cnt + jnp.where(v <= _R2, 1, 0).astype(jnp.int32)

    cnt_ref[:] = lax.fori_loop(0, _K, body, jnp.zeros((bm, 1), jnp.int32))


def _select(pos_c, pos, interpret=False):
    m = pos_c.shape[0]
    n = pos.shape[0]
    bm = 8
    grid = m // bm
    kern = pl.pallas_call(
        functools.partial(_select_body, n=n),
        grid=(grid,),
        in_specs=[
            pl.BlockSpec((bm, 1), lambda g: (g, 0)),
            pl.BlockSpec((bm, 1), lambda g: (g, 0)),
            pl.BlockSpec((bm, 1), lambda g: (g, 0)),
            pl.BlockSpec((1, n), lambda g: (0, 0)),
            pl.BlockSpec((1, n), lambda g: (0, 0)),
            pl.BlockSpec((1, n), lambda g: (0, 0)),
        ],
        out_specs=(pl.BlockSpec((1, _K, bm), lambda g: (g, 0, 0)),
                   pl.BlockSpec((bm, 1), lambda g: (g, 0))),
        out_shape=(jax.ShapeDtypeStruct((grid, _K, bm), jnp.int32),
                   jax.ShapeDtypeStruct((m, 1), jnp.int32)),
        scratch_shapes=[pltpu.VMEM((bm, n), jnp.float32)],
        interpret=interpret,
    )
    nbrt, cnt = kern(pos_c[:, 0:1], pos_c[:, 1:2], pos_c[:, 2:3],
                     pos[:, 0].reshape(1, n), pos[:, 1].reshape(1, n),
                     pos[:, 2].reshape(1, n))
    return jnp.transpose(nbrt, (0, 2, 1)).reshape(m, _K), cnt


_NC = 2
_NSUB = 16
_NW = _NC * _NSUB
_LN = 16
_SCK = 128


def _wid():
    return lax.axis_index("s") * _NC + lax.axis_index("c")


def _b16i(v):
    u = plsc.bitcast(v, jnp.uint32)
    bias = ((u >> 16) & jnp.uint32(1)) + jnp.uint32(0x7FFF)
    u2 = (u + bias) & jnp.uint32(0xFFFF0000)
    return plsc.bitcast(u2, jnp.float32)


def _splat(ref, i):
    base = (i // _LN) * _LN
    v = ref[pl.ds(base, _LN)]
    lanes = lax.iota(jnp.int32, _LN)
    s = jnp.sum(jnp.where(lanes == i - base, v, jnp.zeros_like(v)))
    return jnp.full((_LN,), s)


def _store1(ref, i, v, dtype):
    base = (i // _LN) * _LN
    lanes = lax.iota(jnp.int32, _LN)
    old = ref[pl.ds(base, _LN)]
    ref[pl.ds(base, _LN)] = jnp.where(lanes == i - base,
                                      jnp.full((_LN,), v, dtype), old)


def _sc_select_body(cx_hbm, cy_hbm, cz_hbm, px_hbm, py_hbm, pz_hbm,
                    nbr_hbm, cnt_hbm,
                    pxv, pyv, pzv, spv, d2v, sv, nbrv, cntv, cxv, cyv, czv,
                    *, n, rpw):
    base_r = _wid() * rpw
    pltpu.sync_copy(px_hbm, pxv)
    pltpu.sync_copy(py_hbm, pyv)
    pltpu.sync_copy(pz_hbm, pzv)
    pltpu.sync_copy(cx_hbm.at[pl.ds(base_r, rpw)], cxv)
    pltpu.sync_copy(cy_hbm.at[pl.ds(base_r, rpw)], cyv)
    pltpu.sync_copy(cz_hbm.at[pl.ds(base_r, rpw)], czv)
    lanes = lax.iota(jnp.int32, _LN)
    nsc = n // _SCK
    bigf = jnp.float32(3e38)
    bigi = jnp.int32(2**30)

    def pre_body(t, _):
        off = t * _LN
        x = pxv[pl.ds(off, _LN)]
        y = pyv[pl.ds(off, _LN)]
        z = pzv[pl.ds(off, _LN)]
        spv[pl.ds(off, _LN)] = (x * x + y * y) + z * z
        pxv[pl.ds(off, _LN)] = _b16i(x)
        pyv[pl.ds(off, _LN)] = _b16i(y)
        pzv[pl.ds(off, _LN)] = _b16i(z)
        return 0
    lax.fori_loop(0, n // _LN, pre_body, 0)

    def row_body(r, _):
        cx = _splat(cxv, r)
        cy = _splat(cyv, r)
        cz = _splat(czv, r)
        sc = (cx * cx + cy * cy) + cz * cz
        cxb = _b16i(cx)
        cyb = _b16i(cy)
        czb = _b16i(cz)

        def sc_body(s, _):
            def t_body(t, m):
                off = s * _SCK + t * _LN
                dot = ((cxb * pxv[pl.ds(off, _LN)] + cyb * pyv[pl.ds(off, _LN)])
                       + czb * pzv[pl.ds(off, _LN)])
                d2 = jnp.maximum((sc + spv[pl.ds(off, _LN)]) - 2.0 * dot, 0.0)
                d2v[pl.ds(off, _LN)] = d2
                return jnp.minimum(m, d2)
            m = lax.fori_loop(0, _SCK // _LN, t_body,
                              jnp.full((_LN,), bigf, jnp.float32))
            _store1(sv, s, jnp.min(m), jnp.float32)
            return 0
        lax.fori_loop(0, nsc, sc_body, 0)

        nvs = nsc // _LN

        def k_body(k, cnt):
            svs = [sv[pl.ds(v * _LN, _LN)] for v in range(nvs)]
            m = svs[0]
            for v in range(1, nvs):
                m = jnp.minimum(m, svs[v])
            s_star = jnp.min(m)
            ssv = jnp.full((_LN,), s_star, jnp.float32)
            cand = jnp.full((_LN,), bigi, jnp.int32)
            for v in range(nvs):
                cand = jnp.minimum(
                    cand, jnp.where(svs[v] == ssv, lanes + v * _LN, bigi))
            sc_star = jnp.min(cand)
            base = sc_star * _SCK

            def f_body(t, carry):
                cand, m1, m2 = carry
                d2 = d2v[pl.ds(base + t * _LN, _LN)]
                cand = jnp.minimum(
                    cand, jnp.where(d2 == ssv, t * _LN + lanes, bigi))
                lt = d2 < m1
                m2 = jnp.where(lt, m1, jnp.minimum(m2, d2))
                m1 = jnp.where(lt, d2, m1)
                return (cand, m1, m2)
            cand, m1, m2 = lax.fori_loop(
                0, _SCK // _LN, f_body,
                (jnp.full((_LN,), bigi, jnp.int32),
                 jnp.full((_LN,), bigf, jnp.float32),
                 jnp.full((_LN,), bigf, jnp.float32)))
            off_star = jnp.min(cand)
            j_star = base + off_star
            _store1(d2v, j_star, bigf, jnp.float32)
            lane_star = off_star - (off_star // _LN) * _LN
            s_new = jnp.min(jnp.where(lanes == lane_star, m2, m1))
            _store1(sv, sc_star, s_new, jnp.float32)
            _store1(nbrv, r * _K + k, j_star, jnp.int32)
            return cnt + jnp.where(s_star <= _R2, 1, 0).astype(jnp.int32)

        cnt = lax.fori_loop(0, _K, k_body, jnp.int32(0))
        _store1(cntv, r, cnt, jnp.int32)
        return 0

    lax.fori_loop(0, rpw, row_body, 0)
    pltpu.sync_copy(nbrv, nbr_hbm.at[pl.ds(base_r * _K, rpw * _K)])
    pltpu.sync_copy(cntv, cnt_hbm.at[pl.ds(base_r, rpw)])


def _sc_select(pos_c, pos):
    m = pos_c.shape[0]
    n = pos.shape[0]
    rpw = m // _NW
    mesh = plsc.VectorSubcoreMesh(core_axis_name="c", subcore_axis_name="s", num_cores=_NC, num_subcores=_NSUB)
    kern = pl.kernel(
        functools.partial(_sc_select_body, n=n, rpw=rpw),
        out_type=(jax.ShapeDtypeStruct((m * _K,), jnp.int32),
                  jax.ShapeDtypeStruct((m,), jnp.int32)),
        mesh=mesh,
        compiler_params=pltpu.CompilerParams(needs_layout_passes=False),
        scratch_types=[
            pltpu.VMEM((n,), jnp.float32),
            pltpu.VMEM((n,), jnp.float32),
            pltpu.VMEM((n,), jnp.float32),
            pltpu.VMEM((n,), jnp.float32),
            pltpu.VMEM((n,), jnp.float32),
            pltpu.VMEM((n // _SCK,), jnp.float32),
            pltpu.VMEM((rpw * _K,), jnp.int32),
            pltpu.VMEM((rpw,), jnp.int32),
            pltpu.VMEM((rpw,), jnp.float32),
            pltpu.VMEM((rpw,), jnp.float32),
            pltpu.VMEM((rpw,), jnp.float32),
        ],
    )
    nbr, cnt = kern(pos_c[:, 0].ravel(), pos_c[:, 1].ravel(),
                    pos_c[:, 2].ravel(),
                    pos[:, 0].ravel(), pos[:, 1].ravel(), pos[:, 2].ravel())
    return nbr.reshape(m, _K), cnt.reshape(m, 1)


def _sc_gather_body(a_hbm, idx_hbm, out_hbm, idxv, rowsv, sem, *, bpw, ck):
    base = _wid() * bpw
    pltpu.sync_copy(idx_hbm.at[pl.ds(base, bpw)], idxv)

    def c_body(i, _):
        off = i * ck
        pltpu.async_copy(a_hbm.at[idxv.at[pl.ds(off, ck)]], rowsv, sem).wait()
        pltpu.sync_copy(rowsv, out_hbm.at[pl.ds(base + off, ck)])
        return 0
    lax.fori_loop(0, bpw // ck, c_body, 0)


def _sc_gather(a, idx):
    b = idx.shape[0]
    d = a.shape[1]
    bpw = b // _NW
    ck = 128
    mesh = plsc.VectorSubcoreMesh(core_axis_name="c", subcore_axis_name="s", num_cores=_NC, num_subcores=_NSUB)
    kern = pl.kernel(
        functools.partial(_sc_gather_body, bpw=bpw, ck=ck),
        out_type=jax.ShapeDtypeStruct((b, d), jnp.float32),
        mesh=mesh,
        compiler_params=pltpu.CompilerParams(needs_layout_passes=False,
                                             use_tc_tiling_on_sc=False),
        scratch_types=[
            pltpu.VMEM((bpw,), jnp.int32),
            pltpu.VMEM((ck, d), jnp.float32),
            pltpu.SemaphoreType.DMA,
        ],
    )
    return kern(a, idx)


def _conv_body(g_ref, c_ref, cnt_ref, w2_ref, b2_ref, out_ref):
    bm = c_ref.shape[0]
    dh = c_ref.shape[1]
    do = w2_ref.shape[1]
    g = g_ref[:].reshape(bm, _K, dh)
    h1 = jnp.maximum(g - c_ref[:][:, None, :], 0.0)
    h2 = jnp.dot(_b16(h1.reshape(bm * _K, dh)), _b16(w2_ref[:]),
                 preferred_element_type=jnp.float32) + b2_ref[:]
    h2 = jnp.maximum(h2, 0.0)
    rowio = lax.broadcasted_iota(jnp.int32, (bm * _K, 1), 0)
    slot = rowio - _K * (rowio // _K)
    pen = jnp.where(slot < cnt_ref[:], 0.0, -jnp.float32(3e38))
    h2 = h2 + pen
    out_ref[:] = jnp.max(h2.reshape(bm, _K, do), axis=1)


def _conv(g, c, cnt, W2, b2, interpret=False):
    m, dh = c.shape
    do = W2.shape[1]
    bm = 32
    grid = m // bm
    return pl.pallas_call(
        _conv_body,
        grid=(grid,),
        in_specs=[
            pl.BlockSpec((bm * _K, dh), lambda g: (g, 0)),
            pl.BlockSpec((bm, dh), lambda g: (g, 0)),
            pl.BlockSpec((bm * _K, 1), lambda g: (g, 0)),
            pl.BlockSpec((dh, do), lambda g: (0, 0)),
            pl.BlockSpec((1, do), lambda g: (0, 0)),
        ],
        out_specs=pl.BlockSpec((bm, do), lambda g: (g, 0)),
        out_shape=jax.ShapeDtypeStruct((m, do), jnp.float32),
        interpret=interpret,
    )(g, c, jnp.repeat(cnt, _K, axis=0), W2, b2.reshape(1, do))


def _run(x, pos, batch, W1, b1, W2, b2, interpret=False):
    n = x.shape[0]
    m = int(n * _RATIO)
    idx = _fps(pos, m, interpret=interpret)
    pos_c = jnp.take(pos, idx, axis=0)
    a, c = _features(x, pos, pos_c, W1, b1, interpret=interpret)
    if interpret:
        nbr, cnt = _select(pos_c, pos, interpret=True)
        g = jnp.take(a, nbr.reshape(-1), axis=0)
    else:
        nbr, cnt = _sc_select(pos_c, pos)
        g = _sc_gather(a, nbr.reshape(-1))
    out = _conv(g, c, cnt, W2, b2, interpret=interpret)
    return (out, pos_c, jnp.take(batch, idx))


def kernel(x, pos, batch, W1, b1, W2, b2):
    return _run(x, pos, batch, W1, b1, W2, b2, interpret=False)


# speedup vs baseline: 1.1447x; 1.1447x over previous
import functools
import jax
import jax.numpy as jnp
from jax import lax
from jax.experimental import pallas as pl
from jax.experimental.pallas import tpu as pltpu
from jax.experimental.pallas import tpu_sc as plsc

_RATIO = 0.25
_R2 = 0.2 * 0.2
_K = 64


def _b16(v):
    return v.astype(jnp.bfloat16).astype(jnp.float32)


def _fps_body(px_ref, py_ref, pz_ref, idx_ref, dists_ref, rm_ref, *, m):
    px = px_ref[:]
    py = py_ref[:]
    pz = pz_ref[:]
    rows, cols = px.shape
    iota_r = lax.broadcasted_iota(jnp.int32, (rows, cols), 0)
    iota_c = lax.broadcasted_iota(jnp.int32, (rows, cols), 1)
    flat = iota_r * cols + iota_c

    px0 = px_ref[0, 0]
    py0 = py_ref[0, 0]
    pz0 = pz_ref[0, 0]
    d0 = (px - px0) ** 2 + (py - py0) ** 2 + (pz - pz0) ** 2
    dists_ref[:] = d0
    idx_ref[0:1, 0:1] = jnp.zeros((1, 1), jnp.int32)

    def body(i, _):
        d = dists_ref[:]
        mx = jnp.max(d)
        cand = jnp.where(d == mx, flat, jnp.int32(2**30))
        nxt = jnp.min(cand)
        r = nxt // cols
        c = nxt - r * cols
        cmask = lax.broadcasted_iota(jnp.int32, (1, cols), 1) == c
        pxv = jnp.sum(jnp.where(cmask, px_ref[pl.ds(r, 1), :], 0.0))
        pyv = jnp.sum(jnp.where(cmask, py_ref[pl.ds(r, 1), :], 0.0))
        pzv = jnp.sum(jnp.where(cmask, pz_ref[pl.ds(r, 1), :], 0.0))
        dn = (px - pxv) ** 2 + (py - pyv) ** 2 + (pz - pzv) ** 2
        dists_ref[:] = jnp.minimum(d, dn)
        idx_ref[pl.ds(i, 1), :] = nxt.reshape(1, 1)
        return 0

    lax.fori_loop(1, m, body, 0)


def _fps(pos, m, interpret=False):
    n = pos.shape[0]
    rows = n // 128
    px = pos[:, 0].reshape(rows, 128)
    py = pos[:, 1].reshape(rows, 128)
    pz = pos[:, 2].reshape(rows, 128)
    idx = pl.pallas_call(
        functools.partial(_fps_body, m=m),
        out_shape=jax.ShapeDtypeStruct((m, 1), jnp.int32),
        scratch_shapes=[pltpu.VMEM((rows, 128), jnp.float32),
                        pltpu.VMEM((rows, 1), jnp.float32)],
        interpret=interpret,
    )(px, py, pz)
    return idx.reshape(m)


def _feat_body(p_ref, q_ref, w_ref, b_ref, a_ref, c_ref):
    w = _b16(w_ref[:])
    a_ref[:] = jnp.dot(_b16(p_ref[:]), w,
                       preferred_element_type=jnp.float32) + b_ref[:]
    c_ref[:] = jnp.dot(_b16(q_ref[:]), w, preferred_element_type=jnp.float32)


def _features(x, pos, pos_c, W1, b1, interpret=False):
    n = x.shape[0]
    m = pos_c.shape[0]
    dh = W1.shape[1]
    p = jnp.concatenate([x, pos, jnp.zeros((n, 128 - 9), jnp.float32)], axis=1)
    q = jnp.concatenate(
        [jnp.zeros((m, 6), jnp.float32), pos_c, jnp.zeros((m, 128 - 9), jnp.float32)],
        axis=1)
    w = jnp.concatenate([W1, jnp.zeros((128 - 9, dh), jnp.float32)], axis=0)
    bn = 1024
    bm = bn * m // n
    grid = n // bn
    return pl.pallas_call(
        _feat_body,
        grid=(grid,),
        in_specs=[
            pl.BlockSpec((bn, 128), lambda g: (g, 0)),
            pl.BlockSpec((bm, 128), lambda g: (g, 0)),
            pl.BlockSpec((128, dh), lambda g: (0, 0)),
            pl.BlockSpec((1, dh), lambda g: (0, 0)),
        ],
        out_specs=(pl.BlockSpec((bn, dh), lambda g: (g, 0)),
                   pl.BlockSpec((bm, dh), lambda g: (g, 0))),
        out_shape=(jax.ShapeDtypeStruct((n, dh), jnp.float32),
                   jax.ShapeDtypeStruct((m, dh), jnp.float32)),
        interpret=interpret,
    )(p, q, w, b1.reshape(1, dh))


def _select_body(cx_ref, cy_ref, cz_ref, px_ref, py_ref, pz_ref,
                 nbrt_ref, cnt_ref, d2_ref, *, n):
    bm = cx_ref.shape[0]
    cx = cx_ref[:]
    cy = cy_ref[:]
    cz = cz_ref[:]
    px = px_ref[:]
    py = py_ref[:]
    pz = pz_ref[:]
    sc = (cx * cx + cy * cy) + cz * cz
    sp = (px * px + py * py) + pz * pz
    dot = (_b16(cx) * _b16(px) + _b16(cy) * _b16(py)) + _b16(cz) * _b16(pz)
    d2_ref[:] = jnp.maximum((sc + sp) - 2.0 * dot, 0.0)
    colv = lax.broadcasted_iota(jnp.int32, (bm, n), 1)
    big = jnp.float32(3e38)

    def body(k, cnt):
        d2 = d2_ref[:]
        v = jnp.min(d2, axis=1, keepdims=True)
        j = jnp.min(jnp.where(d2 == v, colv, jnp.int32(2**30)), axis=1,
                    keepdims=True)
        nbrt_ref[0:1, pl.ds(k, 1), :] = j.reshape(1, 1, bm)
        d2_ref[:] = jnp.where(colv == j, big, d2)
        return cnt + jnp.where(v <= _R2, 1, 0).astype(jnp.int32)

    cnt_ref[:] = lax.fori_loop(0, _K, body, jnp.zeros((bm, 1), jnp.int32))


def _select(pos_c, pos, interpret=False):
    m = pos_c.shape[0]
    n = pos.shape[0]
    bm = 8
    grid = m // bm
    kern = pl.pallas_call(
        functools.partial(_select_body, n=n),
        grid=(grid,),
        in_specs=[
            pl.BlockSpec((bm, 1), lambda g: (g, 0)),
            pl.BlockSpec((bm, 1), lambda g: (g, 0)),
            pl.BlockSpec((bm, 1), lambda g: (g, 0)),
            pl.BlockSpec((1, n), lambda g: (0, 0)),
            pl.BlockSpec((1, n), lambda g: (0, 0)),
            pl.BlockSpec((1, n), lambda g: (0, 0)),
        ],
        out_specs=(pl.BlockSpec((1, _K, bm), lambda g: (g, 0, 0)),
                   pl.BlockSpec((bm, 1), lambda g: (g, 0))),
        out_shape=(jax.ShapeDtypeStruct((grid, _K, bm), jnp.int32),
                   jax.ShapeDtypeStruct((m, 1), jnp.int32)),
        scratch_shapes=[pltpu.VMEM((bm, n), jnp.float32)],
        interpret=interpret,
    )
    nbrt, cnt = kern(pos_c[:, 0:1], pos_c[:, 1:2], pos_c[:, 2:3],
                     pos[:, 0].reshape(1, n), pos[:, 1].reshape(1, n),
                     pos[:, 2].reshape(1, n))
    return jnp.transpose(nbrt, (0, 2, 1)).reshape(m, _K), cnt


_NC = 2
_NSUB = 16
_NW = _NC * _NSUB
_LN = 16
_SCK = 256


def _wid():
    return lax.axis_index("s") * _NC + lax.axis_index("c")


def _b16i(v):
    u = plsc.bitcast(v, jnp.uint32)
    bias = ((u >> 16) & jnp.uint32(1)) + jnp.uint32(0x7FFF)
    u2 = (u + bias) & jnp.uint32(0xFFFF0000)
    return plsc.bitcast(u2, jnp.float32)


def _splat(ref, i):
    base = (i // _LN) * _LN
    v = ref[pl.ds(base, _LN)]
    lanes = lax.iota(jnp.int32, _LN)
    s = jnp.sum(jnp.where(lanes == i - base, v, jnp.zeros_like(v)))
    return jnp.full((_LN,), s)


def _store1(ref, i, v, dtype):
    base = (i // _LN) * _LN
    lanes = lax.iota(jnp.int32, _LN)
    old = ref[pl.ds(base, _LN)]
    ref[pl.ds(base, _LN)] = jnp.where(lanes == i - base,
                                      jnp.full((_LN,), v, dtype), old)


def _sc_select_body(cx_hbm, cy_hbm, cz_hbm, px_hbm, py_hbm, pz_hbm,
                    nbr_hbm, cnt_hbm,
                    pxv, pyv, pzv, spv, d2v, sv, nbrv, cntv, cxv, cyv, czv,
                    *, n, rpw):
    base_r = _wid() * rpw
    pltpu.sync_copy(px_hbm, pxv)
    pltpu.sync_copy(py_hbm, pyv)
    pltpu.sync_copy(pz_hbm, pzv)
    pltpu.sync_copy(cx_hbm.at[pl.ds(base_r, rpw)], cxv)
    pltpu.sync_copy(cy_hbm.at[pl.ds(base_r, rpw)], cyv)
    pltpu.sync_copy(cz_hbm.at[pl.ds(base_r, rpw)], czv)
    lanes = lax.iota(jnp.int32, _LN)
    nsc = n // _SCK
    bigf = jnp.float32(3e38)
    bigi = jnp.int32(2**30)

    def pre_body(t, _):
        off = t * _LN
        x = pxv[pl.ds(off, _LN)]
        y = pyv[pl.ds(off, _LN)]
        z = pzv[pl.ds(off, _LN)]
        spv[pl.ds(off, _LN)] = (x * x + y * y) + z * z
        pxv[pl.ds(off, _LN)] = _b16i(x)
        pyv[pl.ds(off, _LN)] = _b16i(y)
        pzv[pl.ds(off, _LN)] = _b16i(z)
        return 0
    lax.fori_loop(0, n // _LN, pre_body, 0)

    def row_body(r, _):
        cx = _splat(cxv, r)
        cy = _splat(cyv, r)
        cz = _splat(czv, r)
        sc = (cx * cx + cy * cy) + cz * cz
        cxb = _b16i(cx)
        cyb = _b16i(cy)
        czb = _b16i(cz)

        def sc_body(s, _):
            def t_body(t, m):
                off = s * _SCK + t * _LN
                dot = ((cxb * pxv[pl.ds(off, _LN)] + cyb * pyv[pl.ds(off, _LN)])
                       + czb * pzv[pl.ds(off, _LN)])
                d2 = jnp.maximum((sc + spv[pl.ds(off, _LN)]) - 2.0 * dot, 0.0)
                d2v[pl.ds(off, _LN)] = d2
                return jnp.minimum(m, d2)
            m = lax.fori_loop(0, _SCK // _LN, t_body,
                              jnp.full((_LN,), bigf, jnp.float32))
            _store1(sv, s, jnp.min(m), jnp.float32)
            return 0
        lax.fori_loop(0, nsc, sc_body, 0)

        nvs = nsc // _LN

        def k_body(k, cnt):
            svs = [sv[pl.ds(v * _LN, _LN)] for v in range(nvs)]
            m = svs[0]
            for v in range(1, nvs):
                m = jnp.minimum(m, svs[v])
            s_star = jnp.min(m)
            ssv = jnp.full((_LN,), s_star, jnp.float32)
            cand = jnp.full((_LN,), bigi, jnp.int32)
            for v in range(nvs):
                cand = jnp.minimum(
                    cand, jnp.where(svs[v] == ssv, lanes + v * _LN, bigi))
            sc_star = jnp.min(cand)
            base = sc_star * _SCK

            def f_body(t, cand):
                d2 = d2v[pl.ds(base + t * _LN, _LN)]
                return jnp.minimum(
                    cand, jnp.where(d2 == ssv, t * _LN + lanes, bigi))
            off_star = jnp.min(lax.fori_loop(
                0, _SCK // _LN, f_body, jnp.full((_LN,), bigi, jnp.int32)))
            j_star = base + off_star
            _store1(d2v, j_star, bigf, jnp.float32)

            def g_body(t, m):
                return jnp.minimum(m, d2v[pl.ds(base + t * _LN, _LN)])
            s_new = jnp.min(lax.fori_loop(
                0, _SCK // _LN, g_body, jnp.full((_LN,), bigf, jnp.float32)))
            _store1(sv, sc_star, s_new, jnp.float32)
            _store1(nbrv, r * _K + k, j_star, jnp.int32)
            return cnt + jnp.where(s_star <= _R2, 1, 0).astype(jnp.int32)

        cnt = lax.fori_loop(0, _K, k_body, jnp.int32(0))
        _store1(cntv, r, cnt, jnp.int32)
        return 0

    lax.fori_loop(0, rpw, row_body, 0)
    pltpu.sync_copy(nbrv, nbr_hbm.at[pl.ds(base_r * _K, rpw * _K)])
    pltpu.sync_copy(cntv, cnt_hbm.at[pl.ds(base_r, rpw)])


def _sc_select(pos_c, pos):
    m = pos_c.shape[0]
    n = pos.shape[0]
    rpw = m // _NW
    mesh = plsc.VectorSubcoreMesh(core_axis_name="c", subcore_axis_name="s", num_cores=_NC, num_subcores=_NSUB)
    kern = pl.kernel(
        functools.partial(_sc_select_body, n=n, rpw=rpw),
        out_type=(jax.ShapeDtypeStruct((m * _K,), jnp.int32),
                  jax.ShapeDtypeStruct((m,), jnp.int32)),
        mesh=mesh,
        compiler_params=pltpu.CompilerParams(needs_layout_passes=False),
        scratch_types=[
            pltpu.VMEM((n,), jnp.float32),
            pltpu.VMEM((n,), jnp.float32),
            pltpu.VMEM((n,), jnp.float32),
            pltpu.VMEM((n,), jnp.float32),
            pltpu.VMEM((n,), jnp.float32),
            pltpu.VMEM((n // _SCK,), jnp.float32),
            pltpu.VMEM((rpw * _K,), jnp.int32),
            pltpu.VMEM((rpw,), jnp.int32),
            pltpu.VMEM((rpw,), jnp.float32),
            pltpu.VMEM((rpw,), jnp.float32),
            pltpu.VMEM((rpw,), jnp.float32),
        ],
    )
    nbr, cnt = kern(pos_c[:, 0].ravel(), pos_c[:, 1].ravel(),
                    pos_c[:, 2].ravel(),
                    pos[:, 0].ravel(), pos[:, 1].ravel(), pos[:, 2].ravel())
    return nbr.reshape(m, _K), cnt.reshape(m, 1)


def _sc_gather_body(a_hbm, idx_hbm, out_hbm, idxv, rowsv, sem, *, bpw, ck):
    base = _wid() * bpw
    pltpu.sync_copy(idx_hbm.at[pl.ds(base, bpw)], idxv)

    def c_body(i, _):
        off = i * ck
        pltpu.async_copy(a_hbm.at[idxv.at[pl.ds(off, ck)]], rowsv, sem).wait()
        pltpu.sync_copy(rowsv, out_hbm.at[pl.ds(base + off, ck)])
        return 0
    lax.fori_loop(0, bpw // ck, c_body, 0)


def _sc_gather(a, idx):
    b = idx.shape[0]
    d = a.shape[1]
    bpw = b // _NW
    ck = 128
    mesh = plsc.VectorSubcoreMesh(core_axis_name="c", subcore_axis_name="s", num_cores=_NC, num_subcores=_NSUB)
    kern = pl.kernel(
        functools.partial(_sc_gather_body, bpw=bpw, ck=ck),
        out_type=jax.ShapeDtypeStruct((b, d), jnp.float32),
        mesh=mesh,
        compiler_params=pltpu.CompilerParams(needs_layout_passes=False,
                                             use_tc_tiling_on_sc=False),
        scratch_types=[
            pltpu.VMEM((bpw,), jnp.int32),
            pltpu.VMEM((ck, d), jnp.float32),
            pltpu.SemaphoreType.DMA,
        ],
    )
    return kern(a, idx)


def _conv_body(g_ref, c_ref, cnt_ref, w2_ref, b2_ref, out_ref):
    bm = c_ref.shape[0]
    dh = c_ref.shape[1]
    do = w2_ref.shape[1]
    g = g_ref[:].reshape(bm, _K, dh)
    h1 = jnp.maximum(g - c_ref[:][:, None, :], 0.0)
    h2 = jnp.dot(_b16(h1.reshape(bm * _K, dh)), _b16(w2_ref[:]),
                 preferred_element_type=jnp.float32) + b2_ref[:]
    h2 = jnp.maximum(h2, 0.0)
    rowio = lax.broadcasted_iota(jnp.int32, (bm * _K, 1), 0)
    slot = rowio - _K * (rowio // _K)
    pen = jnp.where(slot < cnt_ref[:], 0.0, -jnp.float32(3e38))
    h2 = h2 + pen
    out_ref[:] = jnp.max(h2.reshape(bm, _K, do), axis=1)


def _conv(g, c, cnt, W2, b2, interpret=False):
    m, dh = c.shape
    do = W2.shape[1]
    bm = 32
    grid = m // bm
    return pl.pallas_call(
        _conv_body,
        grid=(grid,),
        in_specs=[
            pl.BlockSpec((bm * _K, dh), lambda g: (g, 0)),
            pl.BlockSpec((bm, dh), lambda g: (g, 0)),
            pl.BlockSpec((bm * _K, 1), lambda g: (g, 0)),
            pl.BlockSpec((dh, do), lambda g: (0, 0)),
            pl.BlockSpec((1, do), lambda g: (0, 0)),
        ],
        out_specs=pl.BlockSpec((bm, do), lambda g: (g, 0)),
        out_shape=jax.ShapeDtypeStruct((m, do), jnp.float32),
        interpret=interpret,
    )(g, c, jnp.repeat(cnt, _K, axis=0), W2, b2.reshape(1, do))


def _run(x, pos, batch, W1, b1, W2, b2, interpret=False):
    n = x.shape[0]
    m = int(n * _RATIO)
    idx = _fps(pos, m, interpret=interpret)
    pos_c = jnp.take(pos, idx, axis=0)
    a, c = _features(x, pos, pos_c, W1, b1, interpret=interpret)
    if interpret:
        nbr, cnt = _select(pos_c, pos, interpret=True)
        g = jnp.take(a, nbr.reshape(-1), axis=0)
    else:
        nbr, cnt = _sc_select(pos_c, pos)
        g = _sc_gather(a, nbr.reshape(-1))
    out = _conv(g, c, cnt, W2, b2, interpret=interpret)
    return (out, pos_c, jnp.take(batch, idx))


def kernel(x, pos, batch, W1, b1, W2, b2):
    return _run(x, pos, batch, W1, b1, W2, b2, interpret=False)
